# speedup vs baseline: 2.6178x; 2.6178x over previous
import functools

import jax
import jax.numpy as jnp
from jax import lax
from jax.experimental import pallas as pl
from jax.experimental.pallas import tpu as pltpu
from jax.experimental.pallas import tpu_sc as plsc

D_MODEL = 128
NUM_CORES = 2
NUM_SUBCORES = 16
NUM_WORKERS = NUM_CORES * NUM_SUBCORES
CHUNK = 128


@jax.jit
def _sc_gather(time2d, pe):
    n_chunks, _ = time2d.shape
    b_total = n_chunks * CHUNK
    g_per_w = n_chunks // NUM_WORKERS
    mesh = plsc.VectorSubcoreMesh(core_axis_name="c", subcore_axis_name="s")

    @functools.partial(
        pl.kernel,
        mesh=mesh,
        out_type=jax.ShapeDtypeStruct((b_total, D_MODEL), jnp.float32),
        scratch_types=[
            pltpu.VMEM((g_per_w, CHUNK), jnp.int32),
            pltpu.VMEM((CHUNK, D_MODEL), jnp.float32),
            pltpu.SemaphoreType.DMA,
        ],
    )
    def k(time_hbm, pe_hbm, out_hbm, idx_v, rows_v, sem):
        wid = lax.axis_index("s") * NUM_CORES + lax.axis_index("c")
        base_chunk = wid * g_per_w
        base_row = base_chunk * CHUNK
        pltpu.sync_copy(time_hbm.at[pl.ds(base_chunk, g_per_w)], idx_v)

        def body(g, carry):
            pltpu.async_copy(pe_hbm.at[idx_v.at[g]], rows_v, sem).wait()
            pltpu.sync_copy(
                rows_v, out_hbm.at[pl.ds(base_row + g * CHUNK, CHUNK)]
            )
            return carry

        lax.fori_loop(0, g_per_w, body, 0)

    return k(time2d, pe)


def kernel(time, pe):
    b, s = time.shape
    time2d = time.astype(jnp.int32).reshape(b * s // CHUNK, CHUNK)
    out = _sc_gather(time2d, pe)
    return out.reshape(b, s, D_MODEL)


# speedup vs baseline: 2.6896x; 1.0274x over previous
import functools

import jax
import jax.numpy as jnp
from jax import lax
from jax.experimental import pallas as pl
from jax.experimental.pallas import tpu as pltpu
from jax.experimental.pallas import tpu_sc as plsc

D_MODEL = 128
NUM_CORES = 2
NUM_SUBCORES = 16
NUM_WORKERS = NUM_CORES * NUM_SUBCORES
CHUNK = 64
NBUF = 4


@jax.jit
def _sc_gather(time2d, pe):
    n_chunks, _ = time2d.shape
    b_total = n_chunks * CHUNK
    g_per_w = n_chunks // NUM_WORKERS
    n_super = g_per_w // NBUF
    mesh = plsc.VectorSubcoreMesh(core_axis_name="c", subcore_axis_name="s")

    @functools.partial(
        pl.kernel,
        mesh=mesh,
        out_type=jax.ShapeDtypeStruct((b_total, D_MODEL), jnp.float32),
        scratch_types=[
            pltpu.VMEM((g_per_w, CHUNK), jnp.int32),
            pltpu.VMEM((2, NBUF, CHUNK, D_MODEL), jnp.float32),
            pltpu.SemaphoreType.DMA,
            pltpu.SemaphoreType.DMA,
        ],
    )
    def k(time_hbm, pe_hbm, out_hbm, idx_v, rows_v, gsem, ssem):
        wid = lax.axis_index("s") * NUM_CORES + lax.axis_index("c")
        base_chunk = wid * g_per_w
        base_row = base_chunk * CHUNK
        pltpu.sync_copy(time_hbm.at[pl.ds(base_chunk, g_per_w)], idx_v)

        def fire_gathers(s, half):
            for b in range(NBUF):
                pltpu.async_copy(
                    pe_hbm.at[idx_v.at[s * NBUF + b]],
                    rows_v.at[half, b],
                    gsem,
                )

        def drain_gathers(s, half):
            for b in range(NBUF):
                pltpu.make_async_copy(
                    pe_hbm.at[idx_v.at[s * NBUF + b]],
                    rows_v.at[half, b],
                    gsem,
                ).wait()

        def fire_stores(s, half):
            for b in range(NBUF):
                pltpu.async_copy(
                    rows_v.at[half, b],
                    out_hbm.at[pl.ds(base_row + (s * NBUF + b) * CHUNK, CHUNK)],
                    ssem,
                )

        def drain_stores(s, half):
            for b in range(NBUF):
                pltpu.make_async_copy(
                    rows_v.at[half, b],
                    out_hbm.at[pl.ds(base_row + (s * NBUF + b) * CHUNK, CHUNK)],
                    ssem,
                ).wait()

        fire_gathers(0, 0)

        def body(s, carry):
            half = lax.rem(s, 2)
            drain_gathers(s, half)

            @pl.when(s >= 1)
            def _():
                drain_stores(s - 1, 1 - half)

            @pl.when(s + 1 < n_super)
            def _():
                fire_gathers(s + 1, 1 - half)

            fire_stores(s, half)
            return carry

        lax.fori_loop(0, n_super, body, 0)
        drain_stores(n_super - 1, lax.rem(n_super - 1, 2))

    return k(time2d, pe)


def kernel(time, pe):
    b, s = time.shape
    time2d = time.astype(jnp.int32).reshape(b * s // CHUNK, CHUNK)
    out = _sc_gather(time2d, pe)
    return out.reshape(b, s, D_MODEL)


# speedup vs baseline: 4.0177x; 1.4938x over previous
import functools

import jax
import jax.numpy as jnp
from jax import lax
from jax.experimental import pallas as pl
from jax.experimental.pallas import tpu as pltpu
from jax.experimental.pallas import tpu_sc as plsc

D_MODEL = 128
SEQ = 20
NUM_CORES = 2
NUM_SUBCORES = 16
NUM_WORKERS = NUM_CORES * NUM_SUBCORES
EPC = 4
NBUF = 4


@jax.jit
def _sc_gather(time1d, pe):
    n_rows = time1d.shape[0]
    n_batch = n_rows // SEQ
    rows_per_chunk = EPC * SEQ
    b_per_w = n_batch // NUM_WORKERS
    r_per_w = b_per_w * SEQ
    n_super = b_per_w // (EPC * NBUF)
    mesh = plsc.VectorSubcoreMesh(core_axis_name="c", subcore_axis_name="s")

    @functools.partial(
        pl.kernel,
        mesh=mesh,
        out_type=jax.ShapeDtypeStruct((n_batch, SEQ, D_MODEL), jnp.float32),
        scratch_types=[
            pltpu.VMEM((r_per_w,), jnp.int32),
            pltpu.VMEM((2, NBUF, rows_per_chunk, D_MODEL), jnp.float32),
            pltpu.SemaphoreType.DMA,
            pltpu.SemaphoreType.DMA,
        ],
    )
    def k(time_hbm, pe_hbm, out_hbm, idx_v, rows_v, gsem, ssem):
        wid = lax.axis_index("s") * NUM_CORES + lax.axis_index("c")
        base_b = wid * b_per_w
        pltpu.sync_copy(time_hbm.at[pl.ds(wid * r_per_w, r_per_w)], idx_v)

        def fire_gathers(s, half):
            for b in range(NBUF):
                c = s * NBUF + b
                pltpu.async_copy(
                    pe_hbm.at[idx_v.at[pl.ds(c * rows_per_chunk, rows_per_chunk)]],
                    rows_v.at[half, b],
                    gsem,
                )

        def drain_gathers(s, half):
            for b in range(NBUF):
                c = s * NBUF + b
                pltpu.make_async_copy(
                    pe_hbm.at[idx_v.at[pl.ds(c * rows_per_chunk, rows_per_chunk)]],
                    rows_v.at[half, b],
                    gsem,
                ).wait()

        def fire_stores(s, half):
            for b in range(NBUF):
                for e in range(EPC):
                    bi = base_b + (s * NBUF + b) * EPC + e
                    pltpu.async_copy(
                        rows_v.at[half, b, pl.ds(e * SEQ, SEQ)],
                        out_hbm.at[bi],
                        ssem,
                    )

        def drain_stores(s, half):
            for b in range(NBUF):
                for e in range(EPC):
                    bi = base_b + (s * NBUF + b) * EPC + e
                    pltpu.make_async_copy(
                        rows_v.at[half, b, pl.ds(e * SEQ, SEQ)],
                        out_hbm.at[bi],
                        ssem,
                    ).wait()

        fire_gathers(0, 0)

        def body(s, carry):
            half = lax.rem(s, 2)
            drain_gathers(s, half)

            @pl.when(s >= 1)
            def _():
                drain_stores(s - 1, 1 - half)

            @pl.when(s + 1 < n_super)
            def _():
                fire_gathers(s + 1, 1 - half)

            fire_stores(s, half)
            return carry

        lax.fori_loop(0, n_super, body, 0)
        drain_stores(n_super - 1, lax.rem(n_super - 1, 2))

    return k(time1d, pe)


def kernel(time, pe):
    b, s = time.shape
    time1d = time.astype(jnp.int32).reshape(b * s)
    return _sc_gather(time1d, pe)
